# baseline (speedup 1.0000x reference)
import functools

import jax
import jax.numpy as jnp
from jax import lax
from jax.experimental import pallas as pl
from jax.experimental.pallas import tpu as pltpu
from jax.experimental.pallas import tpu_sc as plsc

D_LOW = 128
D_HIGH = 1024
IDX_CHUNK = 128


@functools.lru_cache(maxsize=None)
def _sc_gather_fn(batch, seq):
    n_tokens = batch * seq
    info = plsc.get_sparse_core_info()
    nw = info.num_cores * info.num_subcores
    b_per_w = n_tokens // nw
    n_chunks = b_per_w // IDX_CHUNK
    per_row = seq // b_per_w
    mesh = plsc.VectorSubcoreMesh(core_axis_name="c", subcore_axis_name="s")

    @functools.partial(
        pl.kernel,
        mesh=mesh,
        out_type=jax.ShapeDtypeStruct((n_tokens, D_LOW), jnp.float32),
        scratch_types=[
            pltpu.VMEM((b_per_w,), jnp.int32),
            pltpu.VMEM((b_per_w, D_LOW), jnp.float32),
            pltpu.SemaphoreType.DMA,
            pltpu.SemaphoreType.DMA,
        ],
    )
    def gather(x_hbm, table_hbm, out_hbm, idx_v, rows_v, gsem, osem):
        wid = lax.axis_index("s") * info.num_cores + lax.axis_index("c")
        row = wid // per_row
        col = (wid % per_row) * b_per_w
        pltpu.sync_copy(x_hbm.at[row, pl.ds(col, b_per_w)], idx_v)
        copies = []
        for j in range(n_chunks):
            copies.append(
                pltpu.async_copy(
                    table_hbm.at[idx_v.at[pl.ds(j * IDX_CHUNK, IDX_CHUNK)]],
                    rows_v.at[pl.ds(j * IDX_CHUNK, IDX_CHUNK)],
                    gsem,
                )
            )
        base = wid * b_per_w
        outs = []
        for j in range(n_chunks):
            copies[j].wait()
            outs.append(
                pltpu.async_copy(
                    rows_v.at[pl.ds(j * IDX_CHUNK, IDX_CHUNK)],
                    out_hbm.at[pl.ds(base + j * IDX_CHUNK, IDX_CHUNK)],
                    osem,
                )
            )
        for c in outs:
            c.wait()

    return gather


def _tc_project(low, w):
    n = low.shape[0]
    blk = 4096

    def body(low_ref, w_ref, out_ref):
        out_ref[...] = jnp.dot(
            low_ref[...], w_ref[...], preferred_element_type=jnp.float32
        )

    return pl.pallas_call(
        body,
        grid=(n // blk,),
        in_specs=[
            pl.BlockSpec((blk, D_LOW), lambda i: (i, 0)),
            pl.BlockSpec((D_LOW, D_HIGH), lambda i: (0, 0)),
        ],
        out_specs=pl.BlockSpec((blk, D_HIGH), lambda i: (i, 0)),
        out_shape=jax.ShapeDtypeStruct((n, D_HIGH), jnp.float32),
    )(low, w)


def kernel(x, embed_table, W):
    b, s = x.shape
    n = b * s
    low = _sc_gather_fn(b, s)(x.astype(jnp.int32), embed_table)
    out = _tc_project(low, W)
    return out.reshape(b, s, D_HIGH)


# speedup vs baseline: 1.0115x; 1.0115x over previous
import functools

import jax
import jax.numpy as jnp
from jax import lax
from jax.experimental import pallas as pl
from jax.experimental.pallas import tpu as pltpu
from jax.experimental.pallas import tpu_sc as plsc

D_LOW = 128
D_HIGH = 1024
IDX_CHUNK = 128


@functools.lru_cache(maxsize=None)
def _sc_gather_fn(batch, seq):
    n_tokens = batch * seq
    info = plsc.get_sparse_core_info()
    nw = info.num_cores * info.num_subcores
    b_per_w = n_tokens // nw
    n_chunks = b_per_w // IDX_CHUNK
    per_row = seq // b_per_w
    mesh = plsc.VectorSubcoreMesh(core_axis_name="c", subcore_axis_name="s")

    @functools.partial(
        pl.kernel,
        mesh=mesh,
        out_type=jax.ShapeDtypeStruct((n_tokens, D_LOW), jnp.float32),
        scratch_types=[
            pltpu.VMEM((b_per_w,), jnp.int32),
            pltpu.VMEM((b_per_w, D_LOW), jnp.float32),
            pltpu.SemaphoreType.DMA,
            pltpu.SemaphoreType.DMA,
        ],
    )
    def gather(x_hbm, table_hbm, out_hbm, idx_v, rows_v, gsem, osem):
        wid = lax.axis_index("s") * info.num_cores + lax.axis_index("c")
        row = wid // per_row
        col = (wid % per_row) * b_per_w
        pltpu.sync_copy(x_hbm.at[row, pl.ds(col, b_per_w)], idx_v)
        copies = []
        for j in range(n_chunks):
            copies.append(
                pltpu.async_copy(
                    table_hbm.at[idx_v.at[pl.ds(j * IDX_CHUNK, IDX_CHUNK)]],
                    rows_v.at[pl.ds(j * IDX_CHUNK, IDX_CHUNK)],
                    gsem,
                )
            )
        base = wid * b_per_w
        outs = []
        for j in range(n_chunks):
            copies[j].wait()
            outs.append(
                pltpu.async_copy(
                    rows_v.at[pl.ds(j * IDX_CHUNK, IDX_CHUNK)],
                    out_hbm.at[pl.ds(base + j * IDX_CHUNK, IDX_CHUNK)],
                    osem,
                )
            )
        for c in outs:
            c.wait()

    return gather


def _tc_project(low, w):
    n = low.shape[0]
    blk = 1024

    def body(low_ref, w_ref, out_ref):
        out_ref[...] = jnp.dot(
            low_ref[...], w_ref[...], preferred_element_type=jnp.float32
        )

    return pl.pallas_call(
        body,
        grid=(n // blk,),
        in_specs=[
            pl.BlockSpec((blk, D_LOW), lambda i: (i, 0)),
            pl.BlockSpec((D_LOW, D_HIGH), lambda i: (0, 0)),
        ],
        out_specs=pl.BlockSpec((blk, D_HIGH), lambda i: (i, 0)),
        out_shape=jax.ShapeDtypeStruct((n, D_HIGH), jnp.float32),
    )(low, w)


def kernel(x, embed_table, W):
    b, s = x.shape
    n = b * s
    low = _sc_gather_fn(b, s)(x.astype(jnp.int32), embed_table)
    out = _tc_project(low, W)
    return out.reshape(b, s, D_HIGH)


# speedup vs baseline: 1.0354x; 1.0236x over previous
import functools

import jax
import jax.numpy as jnp
from jax import lax
from jax.experimental import pallas as pl
from jax.experimental.pallas import tpu as pltpu
from jax.experimental.pallas import tpu_sc as plsc

D_LOW = 128
D_HIGH = 1024
IDX_CHUNK = 128


@functools.lru_cache(maxsize=None)
def _sc_gather_fn(batch, seq):
    n_tokens = batch * seq
    info = plsc.get_sparse_core_info()
    nw = info.num_cores * info.num_subcores
    b_per_w = n_tokens // nw
    n_chunks = b_per_w // IDX_CHUNK
    per_row = seq // b_per_w
    mesh = plsc.VectorSubcoreMesh(core_axis_name="c", subcore_axis_name="s")

    @functools.partial(
        pl.kernel,
        mesh=mesh,
        out_type=jax.ShapeDtypeStruct((n_tokens, D_LOW), jnp.float32),
        scratch_types=[
            pltpu.VMEM((b_per_w,), jnp.int32),
            pltpu.VMEM((b_per_w, D_LOW), jnp.float32),
            pltpu.SemaphoreType.DMA,
        ],
    )
    def gather(x_hbm, table_hbm, out_hbm, idx_v, rows_v, sem):
        wid = lax.axis_index("s") * info.num_cores + lax.axis_index("c")
        row = wid // per_row
        col = (wid % per_row) * b_per_w
        pltpu.sync_copy(x_hbm.at[row, pl.ds(col, b_per_w)], idx_v)
        copies = []
        for j in range(n_chunks):
            copies.append(
                pltpu.async_copy(
                    table_hbm.at[idx_v.at[pl.ds(j * IDX_CHUNK, IDX_CHUNK)]],
                    rows_v.at[pl.ds(j * IDX_CHUNK, IDX_CHUNK)],
                    sem,
                )
            )
        for c in copies:
            c.wait()
        pltpu.sync_copy(rows_v, out_hbm.at[pl.ds(wid * b_per_w, b_per_w)])

    return gather


def _tc_project(low, w):
    n = low.shape[0]
    blk = 2048

    def body(low_ref, w_ref, out_ref):
        out_ref[...] = jnp.dot(
            low_ref[...], w_ref[...], preferred_element_type=jnp.float32
        )

    return pl.pallas_call(
        body,
        grid=(n // blk,),
        in_specs=[
            pl.BlockSpec((blk, D_LOW), lambda i: (i, 0)),
            pl.BlockSpec((D_LOW, D_HIGH), lambda i: (0, 0)),
        ],
        out_specs=pl.BlockSpec((blk, D_HIGH), lambda i: (i, 0)),
        out_shape=jax.ShapeDtypeStruct((n, D_HIGH), jnp.float32),
    )(low, w)


def kernel(x, embed_table, W):
    b, s = x.shape
    n = b * s
    low = _sc_gather_fn(b, s)(x.astype(jnp.int32), embed_table)
    out = _tc_project(low, W)
    return out.reshape(b, s, D_HIGH)
